# Initial kernel scaffold; baseline (speedup 1.0000x reference)
import jax
import jax.numpy as jnp
from jax.experimental import pallas as pl


def kernel(x, rank_emb):
    raise NotImplementedError("write your pallas kernel here")


# speedup vs baseline: 2.8538x; 2.8538x over previous
import jax
import jax.numpy as jnp
from jax.experimental import pallas as pl


_TB = 512


def _add_kernel(x_ref, emb_ref, o_ref):
    o_ref[...] = x_ref[...] + emb_ref[...]


def kernel(x, rank_emb):
    B, T, D = x.shape
    grid = (T // _TB, B)
    return pl.pallas_call(
        _add_kernel,
        grid=grid,
        in_specs=[
            pl.BlockSpec((1, _TB, D), lambda t, b: (b, t, 0)),
            pl.BlockSpec((_TB, D), lambda t, b: (t, 0)),
        ],
        out_specs=pl.BlockSpec((1, _TB, D), lambda t, b: (b, t, 0)),
        out_shape=jax.ShapeDtypeStruct((B, T, D), x.dtype),
    )(x, rank_emb)


# speedup vs baseline: 3.1790x; 1.1140x over previous
import jax
import jax.numpy as jnp
from jax.experimental import pallas as pl


_TB = 1024


def _add_kernel(x_ref, emb_ref, o_ref):
    o_ref[...] = x_ref[...] + emb_ref[...]


def kernel(x, rank_emb):
    B, T, D = x.shape
    grid = (T // _TB, B)
    return pl.pallas_call(
        _add_kernel,
        grid=grid,
        in_specs=[
            pl.BlockSpec((1, _TB, D), lambda t, b: (b, t, 0)),
            pl.BlockSpec((_TB, D), lambda t, b: (t, 0)),
        ],
        out_specs=pl.BlockSpec((1, _TB, D), lambda t, b: (b, t, 0)),
        out_shape=jax.ShapeDtypeStruct((B, T, D), x.dtype),
    )(x, rank_emb)


# speedup vs baseline: 3.3080x; 1.0406x over previous
import jax
import jax.numpy as jnp
from jax.experimental import pallas as pl


_TB = 2048


def _add_kernel(x_ref, emb_ref, o_ref):
    o_ref[...] = x_ref[...] + emb_ref[...]


def kernel(x, rank_emb):
    B, T, D = x.shape
    grid = (T // _TB, B)
    return pl.pallas_call(
        _add_kernel,
        grid=grid,
        in_specs=[
            pl.BlockSpec((1, _TB, D), lambda t, b: (b, t, 0)),
            pl.BlockSpec((_TB, D), lambda t, b: (t, 0)),
        ],
        out_specs=pl.BlockSpec((1, _TB, D), lambda t, b: (b, t, 0)),
        out_shape=jax.ShapeDtypeStruct((B, T, D), x.dtype),
    )(x, rank_emb)
